# baseline (speedup 1.0000x reference)
import jax
import jax.numpy as jnp
from jax import lax
from jax.experimental import pallas as pl
from jax.experimental.pallas import tpu as pltpu
from jax.experimental.pallas import tpu_sc as plsc

N = 50000
M = 25000
NNZ = 200000
D = 256

NUM_CORES = 2
NUM_TILES = 16
LANES = 16

M_PAD = 25600
HALF = M_PAD // NUM_CORES
NPASS = 5
R = HALF // NPASS
RBITS = 12
S = 12800
CH = 2560
NCH = S // CH
G = 64
CAP = S
NBUF = 2
DW = D // 2
WB = R // NUM_TILES
ZR = 8
NZ = WB // ZR


def _pool_body(x_hbm, rows_hbm, cols_hbm, vals_hbm, out_hbm,
               str_, stc_, stv_, cpk, cvals,
               cidx0, ridx0, cidx1, ridx1,
               gb0, gb1, sb0, sb1, zbuf, acc,
               stsem, gsem0, gsem1, ssem0, ssem1, zsem):
    ci = lax.axis_index("c")
    si = lax.axis_index("s")
    ebase = si * S

    zeros16 = jnp.zeros((LANES,), jnp.float32)

    @pl.loop(0, ZR)
    def _(r):
        for j in range(D // LANES):
            zbuf[r, pl.ds(j * LANES, LANES)] = zeros16

    grp_bufs = [(cidx0, ridx0, gb0, sb0, gsem0, ssem0),
                (cidx1, ridx1, gb1, sb1, gsem1, ssem1)]

    lanes_iota = lax.iota(jnp.int32, LANES)

    def fill_idx(g, cidxb, ridxb):
        for u in range(G // LANES):
            sl = pl.ds(u * LANES, LANES)
            pkv = cpk[pl.ds(g * G + u * LANES, LANES)]
            cidxb[sl] = lax.shift_right_logical(pkv, RBITS)
            ridxb[sl] = pkv & ((1 << RBITS) - 1)

    for p in range(NPASS):
        base = ci * HALF + p * R
        obase = base
        rb = si * WB

        zdescs = [pltpu.async_copy(zbuf, acc.at[pl.ds(rb + z * ZR, ZR)], zsem)
                  for z in range(NZ)]

        cnt = jnp.int32(0)
        for k in range(NCH):
            off = ebase + k * CH
            offc = jnp.where(off > NNZ - CH, NNZ - CH, off)
            descs = [pltpu.async_copy(rows_hbm.at[pl.ds(offc, CH)], str_,
                                      stsem),
                     pltpu.async_copy(cols_hbm.at[pl.ds(offc, CH)], stc_,
                                      stsem),
                     pltpu.async_copy(vals_hbm.at[pl.ds(offc, CH)], stv_,
                                      stsem)]
            for dsc in descs:
                dsc.wait()

            def step(i, cnt, off=off, offc=offc):
                rv = str_[pl.ds(i * LANES, LANES)]
                pos = offc + i * LANES + lanes_iota
                m = ((rv >= base) & (rv < base + R)
                     & (pos >= off) & (pos < NNZ))
                lrow = rv - base
                pk = lax.shift_left(stc_[pl.ds(i * LANES, LANES)],
                                    RBITS) | (lrow & ((1 << RBITS) - 1))
                plsc.store_compressed(cpk.at[pl.ds(cnt, LANES)], pk, mask=m)
                plsc.store_compressed(cvals.at[pl.ds(cnt, LANES)],
                                      stv_[pl.ds(i * LANES, LANES)], mask=m)
                return cnt + jnp.sum(m.astype(jnp.int32))

            cnt = lax.fori_loop(0, CH // LANES, step, cnt)

        ngroups = (cnt + G - 1) // G

        def tail_zero(k, carry):
            pos = k * LANES + lanes_iota
            mm = pos >= cnt
            sl = pl.ds(k * LANES, LANES)
            cpk[sl] = jnp.where(mm, 0, cpk[sl])
            cvals[sl] = jnp.where(mm, 0.0, cvals[sl])
            return carry

        lax.fori_loop(cnt // LANES, (ngroups * G) // LANES, tail_zero,
                      jnp.int32(0))

        for dz in zdescs:
            dz.wait()
        plsc.subcore_barrier()

        for i in range(NBUF - 1):
            @pl.when(i < ngroups)
            def _(i=i):
                cidxb, ridxb, gbb, _sb, gsemb, _s = grp_bufs[i]
                fill_idx(jnp.int32(i), cidxb, ridxb)
                pltpu.async_copy(x_hbm.at[cidxb], gbb, gsemb)

        def make_proc(b):
            cidxb, ridxb, gbb, sbb, gsemb, ssemb = grp_bufs[b]
            cidxn, ridxn, gbn, sbn, gsemn, ssemn = grp_bufs[(b + 1) % NBUF]

            def proc(g):
                pltpu.make_async_copy(x_hbm.at[cidxb], gbb, gsemb).wait()

                @pl.when(g > 1)
                def _():
                    pltpu.make_async_copy(out_hbm.at[pl.ds(0, G)], sbb,
                                          ssemb).wait()

                @pl.when(g + 1 < ngroups)
                def _():
                    fill_idx(g + 1, cidxn, ridxn)
                    pltpu.async_copy(x_hbm.at[cidxn], gbn, gsemn)

                def row(e, carry):
                    vsp = plsc.load_gather(
                        cvals, [jnp.full((LANES,), g * G + e, jnp.int32)])
                    for u in range(DW // LANES):
                        w = gbb[e, pl.ds(u * LANES, LANES)]
                        lo = plsc.bitcast(lax.shift_left(w, 16), jnp.float32)
                        hi = plsc.bitcast(w & jnp.int32(-65536), jnp.float32)
                        sbb[e, pl.ds(2 * u * LANES, LANES)] = lo * vsp
                        sbb[e, pl.ds((2 * u + 1) * LANES, LANES)] = hi * vsp
                    return carry

                lax.fori_loop(0, G, row, jnp.int32(0))
                for u in range(G // LANES):
                    idxv = ridxb[pl.ds(u * LANES, LANES)]
                    pltpu.async_copy(sbb.at[pl.ds(u * LANES, LANES)],
                                     acc.at[idxv], ssemb, add=True)

            return proc

        procs = [make_proc(b) for b in range(NBUF)]

        def group(g, carry):
            gm = lax.rem(g, jnp.int32(NBUF))
            for b in range(NBUF):
                @pl.when(gm == b)
                def _(b=b):
                    procs[b](g)
            return carry

        lax.fori_loop(0, ngroups, group, jnp.int32(0))

        for b in range(NBUF):
            cond = jnp.bool_(False)
            for j in range(1, NBUF + 1):
                cond = cond | ((ngroups >= j)
                               & (lax.rem(ngroups - j, jnp.int32(NBUF)) == b))

            @pl.when(cond)
            def _(b=b):
                _c, _r, _gb, sbb, _g, ssemb = grp_bufs[b]
                pltpu.make_async_copy(out_hbm.at[pl.ds(0, G)], sbb,
                                      ssemb).wait()

        plsc.subcore_barrier()

        pltpu.sync_copy(acc.at[pl.ds(rb, WB)],
                        out_hbm.at[pl.ds(obase + rb, WB)])

        if p + 1 < NPASS:
            plsc.subcore_barrier()


_pool = pl.kernel(
    _pool_body,
    out_type=jax.ShapeDtypeStruct((M_PAD, D), jnp.float32),
    mesh=plsc.VectorSubcoreMesh(core_axis_name="c", subcore_axis_name="s",
                                num_cores=NUM_CORES, num_subcores=NUM_TILES),
    compiler_params=pltpu.CompilerParams(use_tc_tiling_on_sc=False,
                                         needs_layout_passes=False),
    scratch_types=[
        pltpu.VMEM((CH,), jnp.int32),
        pltpu.VMEM((CH,), jnp.int32),
        pltpu.VMEM((CH,), jnp.float32),
        pltpu.VMEM((CAP,), jnp.int32),
        pltpu.VMEM((CAP,), jnp.float32),
        pltpu.VMEM((G,), jnp.int32),
        pltpu.VMEM((G,), jnp.int32),
        pltpu.VMEM((G,), jnp.int32),
        pltpu.VMEM((G,), jnp.int32),
        pltpu.VMEM((G, DW), jnp.int32),
        pltpu.VMEM((G, DW), jnp.int32),
        pltpu.VMEM((G, D), jnp.float32),
        pltpu.VMEM((G, D), jnp.float32),
        pltpu.VMEM((ZR, D), jnp.float32),
        pltpu.VMEM_SHARED((R, D), jnp.float32),
        pltpu.SemaphoreType.DMA,
        pltpu.SemaphoreType.DMA,
        pltpu.SemaphoreType.DMA,
        pltpu.SemaphoreType.DMA,
        pltpu.SemaphoreType.DMA,
        pltpu.SemaphoreType.DMA,
    ],
)


def _mm_body(p_ref, w_ref, o_ref):
    o_ref[...] = jnp.maximum(
        jnp.dot(p_ref[...], w_ref[...], preferred_element_type=jnp.float32),
        0.0)


_MM_BLK = 1000


def _matmul(pooled, W):
    return pl.pallas_call(
        _mm_body,
        grid=(M // _MM_BLK,),
        in_specs=[
            pl.BlockSpec((_MM_BLK, D), lambda i: (i, 0)),
            pl.BlockSpec((D, D), lambda i: (0, 0)),
        ],
        out_specs=pl.BlockSpec((_MM_BLK, D), lambda i: (i, 0)),
        out_shape=jax.ShapeDtypeStruct((M, D), jnp.float32),
    )(pooled, W)


_BLK_PERM = [2 * l for l in range(16)] + [2 * l + 1 for l in range(16)]
_COL_PERM = tuple(32 * u + l for u in range(D // 32) for l in _BLK_PERM)


@jax.jit
def kernel(x, u_rows, u_cols, u_vals, W):
    xw = lax.bitcast_convert_type(
        x.astype(jnp.bfloat16).reshape(N, DW, 2), jnp.int32)
    pooled = _pool(xw, u_rows, u_cols, u_vals)
    w_perm = W[jnp.array(_COL_PERM, dtype=jnp.int32), :]
    return _matmul(pooled, w_perm)


# speedup vs baseline: 2.2465x; 2.2465x over previous
import jax
import jax.numpy as jnp
from jax import lax
from jax.experimental import pallas as pl
from jax.experimental.pallas import tpu as pltpu
from jax.experimental.pallas import tpu_sc as plsc

N = 50000
M = 25000
NNZ = 200000
D = 256

NUM_CORES = 2
NUM_TILES = 16
LANES = 16

M_PAD = 25600
HALF = M_PAD // NUM_CORES
NPASS = 4
R = HALF // NPASS
RBITS = 12
S = 12800
CH = 2560
NCH = S // CH
G = 64
CAP = S
NBUF = 2
WB = R // NUM_TILES
ZR = 8
NZ = WB // ZR


def _pool_body(x_hbm, rows_hbm, cols_hbm, vals_hbm, out_hbm,
               str_, stc_, stv_, cpk, cvals,
               cidx0, ridx0, cidx1, ridx1,
               gb0, gb1, zbuf, acc,
               stsem, gsem0, gsem1, ssem0, ssem1, zsem):
    ci = lax.axis_index("c")
    si = lax.axis_index("s")
    ebase = si * S

    zeros16 = jnp.zeros((LANES,), jnp.float32)

    @pl.loop(0, ZR)
    def _(r):
        for j in range(D // LANES):
            zbuf[r, pl.ds(j * LANES, LANES)] = zeros16

    grp_bufs = [(cidx0, ridx0, gb0, gsem0, ssem0),
                (cidx1, ridx1, gb1, gsem1, ssem1)]

    lanes_iota = lax.iota(jnp.int32, LANES)

    def fill_idx(g, cidxb, ridxb):
        for u in range(G // LANES):
            sl = pl.ds(u * LANES, LANES)
            pkv = cpk[pl.ds(g * G + u * LANES, LANES)]
            cidxb[sl] = lax.shift_right_logical(pkv, RBITS)
            ridxb[sl] = pkv & ((1 << RBITS) - 1)

    for p in range(NPASS):
        base = ci * HALF + p * R
        obase = base
        rb = si * WB

        zdescs = [pltpu.async_copy(zbuf, acc.at[pl.ds(rb + z * ZR, ZR)], zsem)
                  for z in range(NZ)]

        cnt = jnp.int32(0)
        for k in range(NCH):
            off = ebase + k * CH
            offc = jnp.where(off > NNZ - CH, NNZ - CH, off)
            descs = [pltpu.async_copy(rows_hbm.at[pl.ds(offc, CH)], str_,
                                      stsem),
                     pltpu.async_copy(cols_hbm.at[pl.ds(offc, CH)], stc_,
                                      stsem),
                     pltpu.async_copy(vals_hbm.at[pl.ds(offc, CH)], stv_,
                                      stsem)]
            for dsc in descs:
                dsc.wait()

            def step(i, cnt, off=off, offc=offc):
                rv = str_[pl.ds(i * LANES, LANES)]
                pos = offc + i * LANES + lanes_iota
                m = ((rv >= base) & (rv < base + R)
                     & (pos >= off) & (pos < NNZ))
                lrow = rv - base
                pk = lax.shift_left(stc_[pl.ds(i * LANES, LANES)],
                                    RBITS) | (lrow & ((1 << RBITS) - 1))
                plsc.store_compressed(cpk.at[pl.ds(cnt, LANES)], pk, mask=m)
                plsc.store_compressed(cvals.at[pl.ds(cnt, LANES)],
                                      stv_[pl.ds(i * LANES, LANES)], mask=m)
                return cnt + jnp.sum(m.astype(jnp.int32))

            cnt = lax.fori_loop(0, CH // LANES, step, cnt)

        ngroups = (cnt + G - 1) // G

        def tail_zero(k, carry):
            pos = k * LANES + lanes_iota
            mm = pos >= cnt
            sl = pl.ds(k * LANES, LANES)
            cpk[sl] = jnp.where(mm, 0, cpk[sl])
            cvals[sl] = jnp.where(mm, 0.0, cvals[sl])
            return carry

        lax.fori_loop(cnt // LANES, (ngroups * G) // LANES, tail_zero,
                      jnp.int32(0))

        for dz in zdescs:
            dz.wait()
        plsc.subcore_barrier()

        for i in range(NBUF - 1):
            @pl.when(i < ngroups)
            def _(i=i):
                cidxb, ridxb, gbb, gsemb, _s = grp_bufs[i]
                fill_idx(jnp.int32(i), cidxb, ridxb)
                pltpu.async_copy(x_hbm.at[cidxb], gbb, gsemb)

        def make_proc(b):
            cidxb, ridxb, gbb, gsemb, ssemb = grp_bufs[b]
            cidxn, ridxn, gbn, gsemn, ssemn = grp_bufs[(b + 1) % NBUF]

            def proc(g):
                pltpu.make_async_copy(x_hbm.at[cidxb], gbb, gsemb).wait()

                @pl.when(g + 1 < ngroups)
                def _():
                    @pl.when(g > 0)
                    def _():
                        pltpu.make_async_copy(x_hbm.at[pl.ds(0, G)], gbn,
                                              ssemn).wait()

                    fill_idx(g + 1, cidxn, ridxn)
                    pltpu.async_copy(x_hbm.at[cidxn], gbn, gsemn)

                def row(e, carry):
                    vsp = plsc.load_gather(
                        cvals, [jnp.full((LANES,), g * G + e, jnp.int32)])
                    for j in range(D // LANES):
                        sl = (e, pl.ds(j * LANES, LANES))
                        gbb[sl] = gbb[sl] * vsp
                    return carry

                lax.fori_loop(0, G, row, jnp.int32(0))
                for u in range(G // LANES):
                    idxv = ridxb[pl.ds(u * LANES, LANES)]
                    pltpu.async_copy(gbb.at[pl.ds(u * LANES, LANES)],
                                     acc.at[idxv], ssemb, add=True)

            return proc

        procs = [make_proc(b) for b in range(NBUF)]

        def group(g, carry):
            gm = lax.rem(g, jnp.int32(NBUF))
            for b in range(NBUF):
                @pl.when(gm == b)
                def _(b=b):
                    procs[b](g)
            return carry

        lax.fori_loop(0, ngroups, group, jnp.int32(0))

        for b in range(NBUF):
            cond = jnp.bool_(False)
            for j in range(1, NBUF + 1):
                cond = cond | ((ngroups >= j)
                               & (lax.rem(ngroups - j, jnp.int32(NBUF)) == b))

            @pl.when(cond)
            def _(b=b):
                _c, _r, gbb, _g, ssemb = grp_bufs[b]
                pltpu.make_async_copy(x_hbm.at[pl.ds(0, G)], gbb,
                                      ssemb).wait()

        plsc.subcore_barrier()

        pltpu.sync_copy(acc.at[pl.ds(rb, WB)],
                        out_hbm.at[pl.ds(obase + rb, WB)])

        if p + 1 < NPASS:
            plsc.subcore_barrier()


_pool = pl.kernel(
    _pool_body,
    out_type=jax.ShapeDtypeStruct((M_PAD, D), jnp.float32),
    mesh=plsc.VectorSubcoreMesh(core_axis_name="c", subcore_axis_name="s",
                                num_cores=NUM_CORES, num_subcores=NUM_TILES),
    compiler_params=pltpu.CompilerParams(use_tc_tiling_on_sc=False,
                                         needs_layout_passes=False),
    scratch_types=[
        pltpu.VMEM((CH,), jnp.int32),
        pltpu.VMEM((CH,), jnp.int32),
        pltpu.VMEM((CH,), jnp.float32),
        pltpu.VMEM((CAP,), jnp.int32),
        pltpu.VMEM((CAP,), jnp.float32),
        pltpu.VMEM((G,), jnp.int32),
        pltpu.VMEM((G,), jnp.int32),
        pltpu.VMEM((G,), jnp.int32),
        pltpu.VMEM((G,), jnp.int32),
        pltpu.VMEM((G, D), jnp.float32),
        pltpu.VMEM((G, D), jnp.float32),
        pltpu.VMEM((ZR, D), jnp.float32),
        pltpu.VMEM_SHARED((R, D), jnp.float32),
        pltpu.SemaphoreType.DMA,
        pltpu.SemaphoreType.DMA,
        pltpu.SemaphoreType.DMA,
        pltpu.SemaphoreType.DMA,
        pltpu.SemaphoreType.DMA,
        pltpu.SemaphoreType.DMA,
    ],
)


def _mm_body(p_ref, w_ref, o_ref):
    o_ref[...] = jnp.maximum(
        jnp.dot(p_ref[...], w_ref[...], preferred_element_type=jnp.float32),
        0.0)


_MM_BLK = 1000


def _matmul(pooled, W):
    return pl.pallas_call(
        _mm_body,
        grid=(M // _MM_BLK,),
        in_specs=[
            pl.BlockSpec((_MM_BLK, D), lambda i: (i, 0)),
            pl.BlockSpec((D, D), lambda i: (0, 0)),
        ],
        out_specs=pl.BlockSpec((_MM_BLK, D), lambda i: (i, 0)),
        out_shape=jax.ShapeDtypeStruct((M, D), jnp.float32),
    )(pooled, W)


@jax.jit
def kernel(x, u_rows, u_cols, u_vals, W):
    pooled = _pool(x, u_rows, u_cols, u_vals)
    return _matmul(pooled, W)
